# baseline (speedup 1.0000x reference)
import functools

import jax
import jax.numpy as jnp
from jax import lax
from jax.experimental import pallas as pl
from jax.experimental.pallas import tpu as pltpu
from jax.experimental.pallas import tpu_sc as plsc

D = 64
B_USER = 16384
B_ITEM = 81920
TOTAL = B_USER + B_ITEM

NC = 2
NS = 16
NW = NC * NS

U_PER_W = B_USER // NW
I_PER_W = B_ITEM // NW
CHUNK = 512
U_CHUNKS = U_PER_W // CHUNK
I_CHUNKS = I_PER_W // CHUNK

_sc_mesh = plsc.VectorSubcoreMesh(core_axis_name="c", subcore_axis_name="s")


@functools.partial(
    pl.kernel,
    out_type=jax.ShapeDtypeStruct((TOTAL, D), jnp.float32),
    mesh=_sc_mesh,
    compiler_params=pltpu.CompilerParams(use_tc_tiling_on_sc=False),
    scratch_types=[
        pltpu.VMEM((CHUNK,), jnp.int32),
        pltpu.VMEM((CHUNK, D), jnp.float32),
        pltpu.SemaphoreType.DMA,
    ],
)
def _sc_gather(user_ids, item_ids, user_table, item_table, out, idx_v, rows_v, sem):
    wid = lax.axis_index("s") * NC + lax.axis_index("c")
    for c in range(U_CHUNKS):
        base = wid * U_PER_W + c * CHUNK
        pltpu.sync_copy(user_ids.at[pl.ds(base, CHUNK)], idx_v)
        pltpu.async_copy(user_table.at[idx_v], rows_v, sem).wait()
        pltpu.sync_copy(rows_v, out.at[pl.ds(base, CHUNK)])
    for c in range(I_CHUNKS):
        base = wid * I_PER_W + c * CHUNK
        pltpu.sync_copy(item_ids.at[pl.ds(base, CHUNK)], idx_v)
        pltpu.async_copy(item_table.at[idx_v], rows_v, sem).wait()
        pltpu.sync_copy(rows_v, out.at[pl.ds(B_USER + base, CHUNK)])


BLK = 1024
UBLKS = B_USER // BLK
NBLKS = TOTAL // BLK


def _tc_body(x_ref, uw_ref, ub_ref, iw_ref, ib_ref, o_ref):
    pid = pl.program_id(0)
    is_user = pid < UBLKS
    w = jnp.where(is_user, uw_ref[...], iw_ref[...])
    b = jnp.where(is_user, ub_ref[...], ib_ref[...])
    x = x_ref[...]
    o_ref[...] = lax.dot_general(
        x, w, (((1,), (1,)), ((), ())), preferred_element_type=jnp.float32
    ) + b


def _tc_project(x, uw, ub, iw, ib):
    return pl.pallas_call(
        _tc_body,
        grid=(NBLKS,),
        in_specs=[
            pl.BlockSpec((BLK, D), lambda i: (i, 0)),
            pl.BlockSpec((D, D), lambda i: (0, 0)),
            pl.BlockSpec((1, D), lambda i: (0, 0)),
            pl.BlockSpec((D, D), lambda i: (0, 0)),
            pl.BlockSpec((1, D), lambda i: (0, 0)),
        ],
        out_specs=pl.BlockSpec((BLK, D), lambda i: (i, 0)),
        out_shape=jax.ShapeDtypeStruct((TOTAL, D), jnp.float32),
    )(x, uw, ub, iw, ib)


def kernel(user_ids, pos_item_ids, neg_item_ids, user_table, item_table,
           user_W, user_b, item_W, item_b):
    item_ids = jnp.concatenate([pos_item_ids, neg_item_ids])
    gathered = _sc_gather(user_ids, item_ids, user_table, item_table)
    out = _tc_project(gathered, user_W, user_b.reshape(1, D),
                      item_W, item_b.reshape(1, D))
    return out[:B_USER], out[B_USER:2 * B_USER], out[2 * B_USER:]


# speedup vs baseline: 1.4887x; 1.4887x over previous
import functools

import jax
import jax.numpy as jnp
from jax import lax
from jax.experimental import pallas as pl
from jax.experimental.pallas import tpu as pltpu
from jax.experimental.pallas import tpu_sc as plsc

D = 64
B_USER = 16384
B_ITEM = 81920
TOTAL = B_USER + B_ITEM

NC = 2
NS = 16
NW = NC * NS

U_PER_W = B_USER // NW
I_PER_W = B_ITEM // NW
CHUNK = 512
U_CHUNKS = U_PER_W // CHUNK
I_CHUNKS = I_PER_W // CHUNK

_sc_mesh = plsc.VectorSubcoreMesh(core_axis_name="c", subcore_axis_name="s")


@functools.partial(
    pl.kernel,
    out_type=jax.ShapeDtypeStruct((TOTAL, D), jnp.float32),
    mesh=_sc_mesh,
    scratch_types=[
        pltpu.VMEM_SHARED((NS, CHUNK), jnp.int32),
        pltpu.SMEM((CHUNK,), jnp.int32),
        pltpu.VMEM((CHUNK, D), jnp.float32),
        pltpu.SemaphoreType.DMA,
    ],
)
def _sc_gather(user_ids, item_ids, user_table, item_table, out, idx_sh, idx_sm,
               rows_v, sem):
    sid = lax.axis_index("s")
    wid = sid * NC + lax.axis_index("c")

    def do_chunk(ids_hbm, ids_base, table, out_base):
        pltpu.sync_copy(ids_hbm.at[pl.ds(ids_base, CHUNK)], idx_sh.at[sid])
        pltpu.sync_copy(idx_sh.at[sid], idx_sm)

        def issue(i, carry):
            row = idx_sm[i]
            pltpu.async_copy(
                table.at[pl.ds(row, 1), :], rows_v.at[pl.ds(i, 1), :], sem
            )
            return carry

        lax.fori_loop(0, CHUNK, issue, 0)
        pltpu.make_async_copy(table.at[pl.ds(0, CHUNK), :], rows_v, sem).wait()
        pltpu.sync_copy(rows_v, out.at[pl.ds(out_base, CHUNK)])

    for c in range(U_CHUNKS):
        base = wid * U_PER_W + c * CHUNK
        do_chunk(user_ids, base, user_table, base)
    for c in range(I_CHUNKS):
        base = wid * I_PER_W + c * CHUNK
        do_chunk(item_ids, base, item_table, B_USER + base)


BLK = 1024
UBLKS = B_USER // BLK
NBLKS = TOTAL // BLK


def _tc_body(x_ref, uw_ref, ub_ref, iw_ref, ib_ref, o_ref):
    pid = pl.program_id(0)
    is_user = pid < UBLKS
    w = jnp.where(is_user, uw_ref[...], iw_ref[...])
    b = jnp.where(is_user, ub_ref[...], ib_ref[...])
    x = x_ref[...]
    o_ref[...] = lax.dot_general(
        x, w, (((1,), (1,)), ((), ())), preferred_element_type=jnp.float32
    ) + b


def _tc_project(x, uw, ub, iw, ib):
    return pl.pallas_call(
        _tc_body,
        grid=(NBLKS,),
        in_specs=[
            pl.BlockSpec((BLK, D), lambda i: (i, 0)),
            pl.BlockSpec((D, D), lambda i: (0, 0)),
            pl.BlockSpec((1, D), lambda i: (0, 0)),
            pl.BlockSpec((D, D), lambda i: (0, 0)),
            pl.BlockSpec((1, D), lambda i: (0, 0)),
        ],
        out_specs=pl.BlockSpec((BLK, D), lambda i: (i, 0)),
        out_shape=jax.ShapeDtypeStruct((TOTAL, D), jnp.float32),
    )(x, uw, ub, iw, ib)


def kernel(user_ids, pos_item_ids, neg_item_ids, user_table, item_table,
           user_W, user_b, item_W, item_b):
    item_ids = jnp.concatenate([pos_item_ids, neg_item_ids])
    gathered = _sc_gather(user_ids, item_ids, user_table, item_table)
    out = _tc_project(gathered, user_W, user_b.reshape(1, D),
                      item_W, item_b.reshape(1, D))
    return out[:B_USER], out[B_USER:2 * B_USER], out[2 * B_USER:]
